# baseline (speedup 1.0000x reference)
import functools

import jax
import jax.numpy as jnp
from jax import lax
from jax.experimental import pallas as pl
from jax.experimental.pallas import tpu as pltpu
from jax.experimental.pallas import tpu_sc as plsc

DIM = 128
B = 16384

_info = plsc.get_sparse_core_info()
NC = _info.num_cores
NS = _info.num_subcores
L = _info.num_lanes
NW = NC * NS

B_PER_W = B // NW
CHUNK = 128
NCHUNK = B_PER_W // CHUNK
NGROUP = CHUNK // 16
RUNROLL = 1


def _sqrt16(x):
    x = x + 1e-24
    i = lax.bitcast_convert_type(x, jnp.int32)
    i = 0x5F3759DF - lax.shift_right_arithmetic(i, 1)
    y = lax.bitcast_convert_type(i, jnp.float32)
    for _ in range(3):
        y = y * (1.5 - 0.5 * x * y * y)
    return x * y


def _compute_chunk(hb, rb, tb, out_v, out_base):
    lane = lax.iota(jnp.int32, 16)

    def row_sum(i):
        accs = []
        for j in range(DIM // 16):
            sl = pl.ds(j * 16, 16)
            df = hb[i, sl] + rb[i, sl] - tb[i, sl]
            accs.append(df * df)
        a = ((accs[0] + accs[1]) + (accs[2] + accs[3])) + (
            (accs[4] + accs[5]) + (accs[6] + accs[7]))
        return jnp.sum(a)

    def group_body(g, _):
        def row_body(rq, packed):
            base = g * 16 + rq * RUNROLL
            for u in range(RUNROLL):
                s = row_sum(base + u)
                packed = jnp.where(lane == rq * RUNROLL + u, s, packed)
            return packed

        packed = lax.fori_loop(0, 16 // RUNROLL, row_body,
                               jnp.zeros((16,), jnp.float32))
        out_v[pl.ds(out_base + g * 16, 16)] = _sqrt16(packed)
        return 0

    lax.fori_loop(0, NGROUP, group_body, 0)


def _make_body():
    mesh = plsc.VectorSubcoreMesh(core_axis_name="c", subcore_axis_name="s")
    scratch = [
        pltpu.VMEM((1, B_PER_W), jnp.int32),
        pltpu.VMEM((1, B_PER_W), jnp.int32),
        pltpu.VMEM((1, B_PER_W), jnp.int32),
        pltpu.VMEM((1, B_PER_W), jnp.int32),
        pltpu.VMEM((1, B_PER_W), jnp.int32),
        pltpu.VMEM((1, B_PER_W), jnp.int32),
        pltpu.VMEM((CHUNK, DIM), jnp.float32),
        pltpu.VMEM((CHUNK, DIM), jnp.float32),
        pltpu.VMEM((CHUNK, DIM), jnp.float32),
        pltpu.VMEM((CHUNK, DIM), jnp.float32),
        pltpu.VMEM((CHUNK, DIM), jnp.float32),
        pltpu.VMEM((CHUNK, DIM), jnp.float32),
        pltpu.VMEM((B_PER_W,), jnp.float32),
        pltpu.VMEM((B_PER_W,), jnp.float32),
        pltpu.SemaphoreType.DMA,
        pltpu.SemaphoreType.DMA,
        pltpu.SemaphoreType.DMA,
    ]

    @functools.partial(
        pl.kernel,
        out_type=(
            jax.ShapeDtypeStruct((B,), jnp.float32),
            jax.ShapeDtypeStruct((B,), jnp.float32),
        ),
        scratch_types=scratch,
        mesh=mesh,
        compiler_params=pltpu.CompilerParams(
            needs_layout_passes=False,
            skip_device_barrier=True, disable_bounds_checks=True,
            disable_semaphore_checks=True),
    )
    def body(ps, ns, ent, rel, pos_out, neg_out,
             ihp, irp, itp, ihn, irn, itn,
             h0, r0, t0, h1, r1, t1, out_p, out_n, sem0, sem1, sem_out):
        wid = lax.axis_index("s") * NC + lax.axis_index("c")
        wbase = wid * B_PER_W
        sl = pl.ds(wbase, B_PER_W)

        hbufs = (h0, h1)
        rbufs = (r0, r1)
        tbufs = (t0, t1)
        sems = (sem0, sem1)
        idxs = ((ihp, irp, itp), (ihn, irn, itn))
        outs = (out_p, out_n)

        idx_cps = [
            pltpu.async_copy(ps.at[pl.ds(0, 1), sl], ihp, sem_out),
            pltpu.async_copy(ps.at[pl.ds(1, 1), sl], irp, sem_out),
            pltpu.async_copy(ps.at[pl.ds(2, 1), sl], itp, sem_out),
            pltpu.async_copy(ns.at[pl.ds(0, 1), sl], ihn, sem_out),
            pltpu.async_copy(ns.at[pl.ds(1, 1), sl], irn, sem_out),
            pltpu.async_copy(ns.at[pl.ds(2, 1), sl], itn, sem_out),
        ]
        for cp in idx_cps:
            cp.wait()

        def start_gathers(k, buf):
            side, c = divmod(k, NCHUNK)
            ih, ir, it = idxs[side]
            isl = pl.ds(c * CHUNK, CHUNK)
            cp_h = pltpu.async_copy(ent.at[ih.at[0, isl]], hbufs[buf], sems[buf])
            cp_r = pltpu.async_copy(rel.at[ir.at[0, isl]], rbufs[buf], sems[buf])
            cp_t = pltpu.async_copy(ent.at[it.at[0, isl]], tbufs[buf], sems[buf])
            return (cp_h, cp_r, cp_t)

        out_cp = None
        pend = start_gathers(0, 0)
        for k in range(2 * NCHUNK):
            side, c = divmod(k, NCHUNK)
            buf = k % 2
            for cp in pend:
                cp.wait()
            if k + 1 < 2 * NCHUNK:
                pend = start_gathers(k + 1, 1 - buf)
            _compute_chunk(hbufs[buf], rbufs[buf], tbufs[buf],
                           outs[side], c * CHUNK)
            if k == NCHUNK - 1:
                out_cp = pltpu.async_copy(out_p, pos_out.at[sl], sem_out)
        pltpu.sync_copy(out_n, neg_out.at[sl])
        out_cp.wait()

    return body


_body = _make_body()


@jax.jit
def kernel(positive_sample, negative_sample, entity_embeddings,
           relation_embeddings):
    pos_dist, neg_dist = _body(positive_sample, negative_sample,
                               entity_embeddings, relation_embeddings)
    return (pos_dist, neg_dist)


# speedup vs baseline: 1.1049x; 1.1049x over previous
import functools

import jax
import jax.numpy as jnp
from jax import lax
from jax.experimental import pallas as pl
from jax.experimental.pallas import tpu as pltpu
from jax.experimental.pallas import tpu_sc as plsc

DIM = 128
B = 16384

_info = plsc.get_sparse_core_info()
NC = _info.num_cores
NS = _info.num_subcores
L = _info.num_lanes
NW = NC * NS

B_PER_W = B // NW
CHUNK = 128
NCHUNK = B_PER_W // CHUNK
NGROUP = CHUNK // 16
RUNROLL = 1


def _sqrt16(x):
    x = x + 1e-24
    i = lax.bitcast_convert_type(x, jnp.int32)
    i = 0x5F3759DF - lax.shift_right_arithmetic(i, 1)
    y = lax.bitcast_convert_type(i, jnp.float32)
    for _ in range(3):
        y = y * (1.5 - 0.5 * x * y * y)
    return x * y


def _compute_chunk(hb, rb, tb, out_v, out_base):
    lane = lax.iota(jnp.int32, 16)

    def row_sum(i):
        accs = []
        for j in range(DIM // 16):
            sl = pl.ds(j * 16, 16)
            df = hb[i, sl] + rb[i, sl] - tb[i, sl]
            accs.append(df * df)
        a = ((accs[0] + accs[1]) + (accs[2] + accs[3])) + (
            (accs[4] + accs[5]) + (accs[6] + accs[7]))
        return jnp.sum(a)

    def group_body(g, _):
        def row_body(rq, packed):
            base = g * 16 + rq * RUNROLL
            for u in range(RUNROLL):
                s = row_sum(base + u)
                packed = jnp.where(lane == rq * RUNROLL + u, s, packed)
            return packed

        packed = lax.fori_loop(0, 16 // RUNROLL, row_body,
                               jnp.zeros((16,), jnp.float32))
        out_v[pl.ds(out_base + g * 16, 16)] = _sqrt16(packed)
        return 0

    lax.fori_loop(0, NGROUP, group_body, 0)


def _make_body():
    mesh = plsc.VectorSubcoreMesh(core_axis_name="c", subcore_axis_name="s")
    scratch = [
        pltpu.VMEM((6, B_PER_W), jnp.int32),
        pltpu.VMEM((CHUNK, DIM), jnp.float32),
        pltpu.VMEM((CHUNK, DIM), jnp.float32),
        pltpu.VMEM((CHUNK, DIM), jnp.float32),
        pltpu.VMEM((CHUNK, DIM), jnp.float32),
        pltpu.VMEM((CHUNK, DIM), jnp.float32),
        pltpu.VMEM((CHUNK, DIM), jnp.float32),
        pltpu.VMEM((2 * B_PER_W,), jnp.float32),
        pltpu.SemaphoreType.DMA,
        pltpu.SemaphoreType.DMA,
        pltpu.SemaphoreType.DMA,
    ]

    @functools.partial(
        pl.kernel,
        out_type=(
            jax.ShapeDtypeStruct((B,), jnp.float32),
            jax.ShapeDtypeStruct((B,), jnp.float32),
        ),
        scratch_types=scratch,
        mesh=mesh,
        compiler_params=pltpu.CompilerParams(
            needs_layout_passes=False, use_tc_tiling_on_sc=False,
            skip_device_barrier=True, disable_bounds_checks=True,
            disable_semaphore_checks=True),
    )
    def body(ps, ns, ent, rel, pos_out, neg_out,
             idx6, h0, r0, t0, h1, r1, t1, out2, sem0, sem1, sem_out):
        wid = lax.axis_index("s") * NC + lax.axis_index("c")
        wbase = wid * B_PER_W
        sl = pl.ds(wbase, B_PER_W)
        nk = 2 * NCHUNK

        hbufs = (h0, h1)
        rbufs = (r0, r1)
        tbufs = (t0, t1)
        sems = (sem0, sem1)

        idx_cps = [
            pltpu.async_copy(ps.at[pl.ds(j, 1), sl],
                             idx6.at[pl.ds(j, 1), :], sem_out)
            for j in range(3)
        ] + [
            pltpu.async_copy(ns.at[pl.ds(j, 1), sl],
                             idx6.at[pl.ds(3 + j, 1), :], sem_out)
            for j in range(3)
        ]
        for cp in idx_cps:
            cp.wait()

        def start_gathers(k, buf):
            side = k // NCHUNK
            c = k - side * NCHUNK
            base = 3 * side
            isl = pl.ds(c * CHUNK, CHUNK)
            pltpu.async_copy(ent.at[idx6.at[base, isl]], hbufs[buf], sems[buf])
            pltpu.async_copy(rel.at[idx6.at[base + 1, isl]], rbufs[buf], sems[buf])
            pltpu.async_copy(ent.at[idx6.at[base + 2, isl]], tbufs[buf], sems[buf])

        def wait_gathers(buf):
            src = ent.at[pl.ds(0, CHUNK)]
            for bufref in (hbufs[buf], rbufs[buf], tbufs[buf]):
                pltpu.make_async_copy(src, bufref, sems[buf]).wait()

        start_gathers(0, 0)
        start_gathers(1, 1)

        def step(kk, _):
            for b in range(2):
                k = 2 * kk + b
                wait_gathers(b)

                @pl.when(k + 2 < nk)
                def _():
                    start_gathers(k + 2, b)

                _compute_chunk(hbufs[b], rbufs[b], tbufs[b], out2, k * CHUNK)

                @pl.when(k == NCHUNK - 1)
                def _():
                    pltpu.async_copy(out2.at[pl.ds(0, B_PER_W)],
                                     pos_out.at[sl], sem_out)
            return 0

        lax.fori_loop(0, NCHUNK, step, 0)
        pltpu.sync_copy(out2.at[pl.ds(B_PER_W, B_PER_W)], neg_out.at[sl])
        pltpu.make_async_copy(out2.at[pl.ds(0, B_PER_W)],
                              pos_out.at[sl], sem_out).wait()

    return body


_body = _make_body()


@jax.jit
def kernel(positive_sample, negative_sample, entity_embeddings,
           relation_embeddings):
    pos_dist, neg_dist = _body(positive_sample, negative_sample,
                               entity_embeddings, relation_embeddings)
    return (pos_dist, neg_dist)
